# baseline (speedup 1.0000x reference)
import jax
import jax.numpy as jnp
from jax import lax
from jax.experimental import pallas as pl
from jax.experimental.pallas import tpu as pltpu
from jax.experimental.pallas import tpu_sc as plsc

B, S, D = 16, 4096, 1024
L = 16
CHUNKS = S // L


def _last_token_body(x_hbm, mask_hbm, out_hbm, mask_v, row_v):
    wid = lax.axis_index("s")

    if True:
        pltpu.sync_copy(mask_hbm.at[wid], mask_v)

        UNROLL = 8
        init = tuple(jnp.zeros((L,), jnp.int32) for _ in range(UNROLL))

        @plsc.parallel_loop(0, CHUNKS // UNROLL, carry=init)
        def accs(i, a):
            return tuple(
                a[j] + mask_v[pl.ds((i * UNROLL + j) * L, L)]
                for j in range(UNROLL)
            )

        acc = accs[0]
        for j in range(1, UNROLL):
            acc = acc + accs[j]
        total = acc[0]
        for lane in range(1, L):
            total = total + acc[lane]
        idx = wid * S + total - 1

        pltpu.sync_copy(x_hbm.at[pl.ds(idx, 1)], row_v)
        pltpu.sync_copy(row_v, out_hbm.at[pl.ds(wid, 1)])


def kernel(x, mask):
    x_flat = x.reshape(B * S, D)
    mask_i = mask.astype(jnp.int32)
    mesh = plsc.VectorSubcoreMesh(core_axis_name="c", subcore_axis_name="s", num_cores=1)
    fn = pl.kernel(
        _last_token_body,
        mesh=mesh,
        out_type=jax.ShapeDtypeStruct((B, D), jnp.float32),
        scratch_types=[
            pltpu.VMEM((S,), jnp.int32),
            pltpu.VMEM((1, D), jnp.float32),
        ],
    )
    return fn(x_flat, mask_i)


# speedup vs baseline: 1.0088x; 1.0088x over previous
import jax
import jax.numpy as jnp
from jax import lax
from jax.experimental import pallas as pl
from jax.experimental.pallas import tpu as pltpu
from jax.experimental.pallas import tpu_sc as plsc

B, S, D = 16, 4096, 1024
L = 16
CHUNKS = S // L


def _last_token_body(x_hbm, mask_hbm, out_hbm, mask_v, row_v, sem0, sem1):
    wid = lax.axis_index("s")

    if True:
        H = S // 2
        cp0 = pltpu.async_copy(mask_hbm.at[wid, pl.ds(0, H)],
                               mask_v.at[pl.ds(0, H)], sem0)
        cp1 = pltpu.async_copy(mask_hbm.at[wid, pl.ds(H, H)],
                               mask_v.at[pl.ds(H, H)], sem1)

        UNROLL = 8
        HCHUNKS = CHUNKS // 2
        init = tuple(jnp.zeros((L,), jnp.int32) for _ in range(UNROLL))

        cp0.wait()

        @plsc.parallel_loop(0, HCHUNKS // UNROLL, carry=init)
        def accs0(i, a):
            return tuple(
                a[j] + mask_v[pl.ds((i * UNROLL + j) * L, L)]
                for j in range(UNROLL)
            )

        cp1.wait()

        @plsc.parallel_loop(0, HCHUNKS // UNROLL, carry=accs0)
        def accs(i, a):
            return tuple(
                a[j] + mask_v[pl.ds(H + (i * UNROLL + j) * L, L)]
                for j in range(UNROLL)
            )

        acc = accs[0]
        for j in range(1, UNROLL):
            acc = acc + accs[j]
        total = acc[0]
        for lane in range(1, L):
            total = total + acc[lane]
        idx = wid * S + total - 1

        pltpu.sync_copy(x_hbm.at[pl.ds(idx, 1)], row_v)
        pltpu.sync_copy(row_v, out_hbm.at[pl.ds(wid, 1)])


def kernel(x, mask):
    x_flat = x.reshape(B * S, D)
    mask_i = mask.astype(jnp.int32)
    mesh = plsc.VectorSubcoreMesh(core_axis_name="c", subcore_axis_name="s", num_cores=1)
    fn = pl.kernel(
        _last_token_body,
        mesh=mesh,
        out_type=jax.ShapeDtypeStruct((B, D), jnp.float32),
        scratch_types=[
            pltpu.VMEM((S,), jnp.int32),
            pltpu.VMEM((1, D), jnp.float32),
            pltpu.SemaphoreType.DMA,
            pltpu.SemaphoreType.DMA,
        ],
    )
    return fn(x_flat, mask_i)


# speedup vs baseline: 1.0119x; 1.0030x over previous
import jax
import jax.numpy as jnp
from jax import lax
from jax.experimental import pallas as pl
from jax.experimental.pallas import tpu as pltpu
from jax.experimental.pallas import tpu_sc as plsc

B, S, D = 16, 4096, 1024
L = 16
CHUNKS = S // L
UNROLL = 8


def _last_token_body(x_hbm, mask_hbm, out_hbm, mask_v, row_v, sem0, sem1):
    wid = lax.axis_index("s")

    H = S // 2
    cp0 = pltpu.async_copy(mask_hbm.at[wid, pl.ds(0, H)],
                           mask_v.at[pl.ds(0, H)], sem0)
    cp1 = pltpu.async_copy(mask_hbm.at[wid, pl.ds(H, H)],
                           mask_v.at[pl.ds(H, H)], sem1)

    HCHUNKS = CHUNKS // 2
    init = tuple(jnp.zeros((L,), jnp.int32) for _ in range(UNROLL))

    cp0.wait()

    @plsc.parallel_loop(0, HCHUNKS // UNROLL, carry=init)
    def accs0(i, a):
        return tuple(
            a[j] + mask_v[pl.ds((i * UNROLL + j) * L, L)]
            for j in range(UNROLL)
        )

    cp1.wait()

    @plsc.parallel_loop(0, HCHUNKS // UNROLL, carry=accs0)
    def accs(i, a):
        return tuple(
            a[j] + mask_v[pl.ds(H + (i * UNROLL + j) * L, L)]
            for j in range(UNROLL)
        )

    acc = accs[0]
    for j in range(1, UNROLL):
        acc = acc + accs[j]
    total = acc[0]
    for lane in range(1, L):
        total = total + acc[lane]
    idx = wid * S + total - 1

    pltpu.sync_copy(x_hbm.at[pl.ds(idx, 1)], row_v)
    pltpu.sync_copy(row_v, out_hbm.at[pl.ds(wid, 1)])


def kernel(x, mask):
    x_flat = x.reshape(B * S, D)
    mask_i = mask.astype(jnp.int32)
    mesh = plsc.VectorSubcoreMesh(
        core_axis_name="c", subcore_axis_name="s", num_cores=1
    )
    fn = pl.kernel(
        _last_token_body,
        mesh=mesh,
        out_type=jax.ShapeDtypeStruct((B, D), jnp.float32),
        scratch_types=[
            pltpu.VMEM((S,), jnp.int32),
            pltpu.VMEM((1, D), jnp.float32),
            pltpu.SemaphoreType.DMA,
            pltpu.SemaphoreType.DMA,
        ],
    )
    return fn(x_flat, mask_i)
